# Initial kernel scaffold; baseline (speedup 1.0000x reference)
import functools

import jax
import jax.numpy as jnp
from jax import lax
from jax.experimental import pallas as pl
from jax.experimental.pallas import tpu as pltpu
from jax.experimental.pallas import tpu_sc as plsc

N = 10000
E = 320000
D = 128
NC = 2
NS = 16
NW = NC * NS
L = 16

G = 79
EPT = G * 128
E_PAD = NW * EPT
NG = 80
N_PAD = NG * 128
TRASH = N

_mesh = plsc.VectorSubcoreMesh(
    core_axis_name="c", subcore_axis_name="s", num_cores=NC, num_subcores=NS)


@functools.partial(
    pl.kernel,
    out_type=jax.ShapeDtypeStruct((NC, 2 * NG, 128), jnp.float32),
    mesh=_mesh,
    scratch_types=[
        pltpu.VMEM((E // NW,), jnp.int32),
        pltpu.VMEM((E // NW,), jnp.int32),
        pltpu.VMEM((NG, 128), jnp.float32),
        pltpu.VMEM((NG, 128), jnp.float32),
        pltpu.VMEM((NG,), jnp.int32),
        pltpu.VMEM((NG,), jnp.int32),
        pltpu.VMEM((16, 128), jnp.float32),
        pltpu.VMEM_SHARED((2 * NG, 128), jnp.float32),
    ],
)
def _sc_degrees(s_hbm, r_hbm, out_hbm, sbuf, rbuf, hs, hr, io_s, io_r, zbuf,
                acc):
    sid = lax.axis_index("s")
    cid = lax.axis_index("c")
    w = sid * NC + cid
    ept = E // NW

    zv = jnp.zeros((L,), jnp.float32)
    ones = jnp.ones((L,), jnp.float32)

    @pl.loop(0, 16)
    def _(i):
        for k in range(8):
            zbuf[i, pl.ds(16 * k, 16)] = zv

    @pl.loop(0, NG)
    def _(i):
        for k in range(8):
            hs[i, pl.ds(16 * k, 16)] = zv
            hr[i, pl.ds(16 * k, 16)] = zv

    for t in range(NG // 16):
        base = lax.iota(jnp.int32, 16) + 16 * t
        io_s[pl.ds(16 * t, 16)] = base
        io_r[pl.ds(16 * t, 16)] = base + NG

    pltpu.sync_copy(zbuf.at[pl.ds(0, 10)], acc.at[pl.ds(sid * 10, 10)])

    pltpu.sync_copy(s_hbm.at[pl.ds(w * ept, ept)], sbuf)
    pltpu.sync_copy(r_hbm.at[pl.ds(w * ept, ept)], rbuf)

    plsc.subcore_barrier()

    @pl.loop(0, ept // L)
    def _(i):
        sv = sbuf[pl.ds(i * L, L)]
        rv = rbuf[pl.ds(i * L, L)]
        plsc.addupdate_scatter(
            hs, [lax.shift_right_logical(sv, 7), lax.bitwise_and(sv, 127)],
            ones)
        plsc.addupdate_scatter(
            hr, [lax.shift_right_logical(rv, 7), lax.bitwise_and(rv, 127)],
            ones)

    pltpu.sync_copy(hs, acc.at[io_s], add=True)
    pltpu.sync_copy(hr, acc.at[io_r], add=True)

    plsc.subcore_barrier()

    @pl.when(sid == 0)
    def _():
        pltpu.sync_copy(acc, out_hbm.at[cid])


@functools.partial(
    pl.kernel,
    out_type=jax.ShapeDtypeStruct((NC, N, D), jnp.float32),
    mesh=_mesh,
    scratch_types=[
        pltpu.VMEM((G, 128), jnp.int32),
        pltpu.VMEM((G, 128), jnp.int32),
        pltpu.VMEM((128, D), jnp.float32),
        pltpu.VMEM((128, D), jnp.float32),
        pltpu.VMEM_SHARED((N_PAD, D), jnp.float32),
        pltpu.SemaphoreType.DMA,
    ],
)
def _sc_segsum(h_hbm, s_hbm, r_hbm, out_hbm, sidx, ridx, rows, zbuf, acc,
               gsem):
    sid = lax.axis_index("s")
    cid = lax.axis_index("c")
    w = sid * NC + cid

    zv = jnp.zeros((L,), jnp.float32)

    @pl.loop(0, 128)
    def _(i):
        for k in range(D // 16):
            zbuf[i, pl.ds(16 * k, 16)] = zv

    for t in range(N_PAD // NS // 128):
        pltpu.sync_copy(zbuf, acc.at[pl.ds(sid * (N_PAD // NS) + t * 128,
                                           128)])

    pltpu.sync_copy(s_hbm.at[w], sidx)
    pltpu.sync_copy(r_hbm.at[w], ridx)

    plsc.subcore_barrier()

    @pl.loop(0, G)
    def _(j):
        pltpu.async_copy(h_hbm.at[sidx.at[j]], rows, gsem).wait()
        pltpu.sync_copy(rows, acc.at[ridx.at[j]], add=True)

    plsc.subcore_barrier()

    rpw = N // NS
    pltpu.sync_copy(acc.at[pl.ds(sid * rpw, rpw)],
                    out_hbm.at[cid, pl.ds(sid * rpw, rpw)])


def _lin_body(x_ref, w_ref, b_ref, d_ref, o_ref):
    h = lax.dot_general(x_ref[...], w_ref[...], (((1,), (1,)), ((), ())),
                        preferred_element_type=jnp.float32)
    h = h + b_ref[...]
    deg = d_ref[0] + d_ref[1]
    o_ref[...] = h * lax.rsqrt(jnp.maximum(deg, 1.0))


def _tc_linear(x, W, b2, ds_p):
    bn = 1000
    return pl.pallas_call(
        _lin_body,
        grid=(N // bn,),
        in_specs=[
            pl.BlockSpec((bn, D), lambda i: (i, 0)),
            pl.BlockSpec((D, D), lambda i: (0, 0)),
            pl.BlockSpec((1, D), lambda i: (0, 0)),
            pl.BlockSpec((NC, bn, 1), lambda i: (0, i, 0)),
        ],
        out_specs=pl.BlockSpec((bn, D), lambda i: (i, 0)),
        out_shape=jax.ShapeDtypeStruct((N, D), jnp.float32),
    )(x, W, b2, ds_p)


def _fin_body(p_ref, d_ref, o_ref):
    o = p_ref[0] + p_ref[1]
    deg = d_ref[0] + d_ref[1]
    o = o * lax.rsqrt(jnp.maximum(deg, 1.0))
    o_ref[...] = o * jax.nn.sigmoid(o)


def _tc_final(outp, dr_p):
    bn = 1000
    return pl.pallas_call(
        _fin_body,
        grid=(N // bn,),
        in_specs=[
            pl.BlockSpec((NC, bn, D), lambda i: (0, i, 0)),
            pl.BlockSpec((NC, bn, 1), lambda i: (0, i, 0)),
        ],
        out_specs=pl.BlockSpec((bn, D), lambda i: (i, 0)),
        out_shape=jax.ShapeDtypeStruct((N, D), jnp.float32),
    )(outp, dr_p)


def kernel(x, adj, W, b):
    s = adj[0].astype(jnp.int32)
    r = adj[1].astype(jnp.int32)

    degp = _sc_degrees(s, r)
    degp = degp.reshape(NC, 2, N_PAD, 1)
    ds_p = degp[:, 0]
    dr_p = degp[:, 1]

    h = _tc_linear(x, W, b.reshape(1, D), ds_p)

    pad = E_PAD - E
    s_p = jnp.concatenate([s, jnp.zeros((pad,), jnp.int32)]).reshape(
        NW, G, 128)
    r_p = jnp.concatenate([r, jnp.full((pad,), TRASH, jnp.int32)]).reshape(
        NW, G, 128)

    outp = _sc_segsum(h, s_p, r_p)
    y = _tc_final(outp, dr_p)
    return (y, adj)


# speedup vs baseline: 3.1187x; 3.1187x over previous
import functools

import jax
import jax.numpy as jnp
from jax import lax
from jax.experimental import pallas as pl
from jax.experimental.pallas import tpu as pltpu
from jax.experimental.pallas import tpu_sc as plsc

N = 10000
E = 320000
D = 128
NC = 2
NS = 16
NW = NC * NS
L = 16

G = 79
EPT = G * 128
E_PAD = NW * EPT
NG = 80
N_PAD = NG * 128
TRASH = N

_mesh = plsc.VectorSubcoreMesh(
    core_axis_name="c", subcore_axis_name="s", num_cores=NC, num_subcores=NS)


@functools.partial(
    pl.kernel,
    out_type=jax.ShapeDtypeStruct((NW * 2 * N_PAD,), jnp.float32),
    mesh=_mesh,
    compiler_params=pltpu.CompilerParams(needs_layout_passes=False),
    scratch_types=[
        pltpu.VMEM((E // NW,), jnp.int32),
        pltpu.VMEM((E // NW,), jnp.int32),
        pltpu.VMEM((N_PAD,), jnp.float32),
        pltpu.VMEM((N_PAD,), jnp.float32),
    ],
)
def _sc_degrees(s_hbm, r_hbm, out_hbm, sbuf, rbuf, hs, hr):
    sid = lax.axis_index("s")
    cid = lax.axis_index("c")
    w = sid * NC + cid
    ept = E // NW

    zv = jnp.zeros((L,), jnp.float32)
    ones = jnp.ones((L,), jnp.float32)

    @pl.loop(0, N_PAD // L)
    def _(i):
        hs[pl.ds(i * L, L)] = zv
        hr[pl.ds(i * L, L)] = zv

    pltpu.sync_copy(s_hbm.at[pl.ds(w * ept, ept)], sbuf)
    pltpu.sync_copy(r_hbm.at[pl.ds(w * ept, ept)], rbuf)

    @pl.loop(0, ept // L)
    def _(i):
        plsc.addupdate_scatter(hs, [sbuf[pl.ds(i * L, L)]], ones)
        plsc.addupdate_scatter(hr, [rbuf[pl.ds(i * L, L)]], ones)

    off = w * 2 * N_PAD
    pltpu.sync_copy(hs, out_hbm.at[pl.ds(off, N_PAD)])
    pltpu.sync_copy(hr, out_hbm.at[pl.ds(off + N_PAD, N_PAD)])


@functools.partial(
    pl.kernel,
    out_type=jax.ShapeDtypeStruct((NC, N, D), jnp.float32),
    mesh=_mesh,
    compiler_params=pltpu.CompilerParams(needs_layout_passes=False),
    scratch_types=[
        pltpu.VMEM((G, 128), jnp.int32),
        pltpu.VMEM((G, 128), jnp.int32),
        pltpu.VMEM((128, D), jnp.float32),
        pltpu.VMEM_SHARED((N_PAD, D), jnp.float32),
        pltpu.SemaphoreType.DMA,
    ],
)
def _sc_segsum(h_hbm, s_hbm, r_hbm, out_hbm, sidx, ridx, rows, acc, gsem):
    sid = lax.axis_index("s")
    cid = lax.axis_index("c")
    w = sid * NC + cid

    zv = jnp.zeros((L,), jnp.float32)

    @pl.loop(0, 128)
    def _(i):
        for k in range(D // 16):
            rows[i, pl.ds(16 * k, 16)] = zv

    for t in range(N_PAD // NS // 128):
        pltpu.sync_copy(rows, acc.at[pl.ds(sid * (N_PAD // NS) + t * 128,
                                           128)])

    pltpu.sync_copy(s_hbm.at[w], sidx)
    pltpu.sync_copy(r_hbm.at[w], ridx)

    plsc.subcore_barrier()

    @pl.loop(0, G)
    def _(j):
        pltpu.async_copy(h_hbm.at[sidx.at[j]], rows, gsem).wait()
        pltpu.sync_copy(rows, acc.at[ridx.at[j]], add=True)

    plsc.subcore_barrier()

    rpw = 624
    pltpu.sync_copy(acc.at[pl.ds(sid * rpw, rpw)],
                    out_hbm.at[cid, pl.ds(sid * rpw, rpw)])

    @pl.when(sid == 0)
    def _():
        tail = NS * rpw
        pltpu.sync_copy(acc.at[pl.ds(tail, N - tail)],
                        out_hbm.at[cid, pl.ds(tail, N - tail)])


def _lin_body(x_ref, w_ref, b_ref, d_ref, o_ref):
    h = lax.dot_general(x_ref[...], w_ref[...], (((1,), (1,)), ((), ())),
                        preferred_element_type=jnp.float32)
    h = h + b_ref[...]
    deg = jnp.sum(d_ref[...], axis=0)
    o_ref[...] = h * lax.rsqrt(jnp.maximum(deg, 1.0))


def _tc_linear(x, W, b2, ds_p):
    bn = 1000
    return pl.pallas_call(
        _lin_body,
        grid=(N // bn,),
        in_specs=[
            pl.BlockSpec((bn, D), lambda i: (i, 0)),
            pl.BlockSpec((D, D), lambda i: (0, 0)),
            pl.BlockSpec((1, D), lambda i: (0, 0)),
            pl.BlockSpec((NW, bn, 1), lambda i: (0, i, 0)),
        ],
        out_specs=pl.BlockSpec((bn, D), lambda i: (i, 0)),
        out_shape=jax.ShapeDtypeStruct((N, D), jnp.float32),
    )(x, W, b2, ds_p)


def _fin_body(p_ref, d_ref, o_ref):
    o = p_ref[0] + p_ref[1]
    deg = jnp.sum(d_ref[...], axis=0)
    o = o * lax.rsqrt(jnp.maximum(deg, 1.0))
    o_ref[...] = o * jax.nn.sigmoid(o)


def _tc_final(outp, dr_p):
    bn = 1000
    return pl.pallas_call(
        _fin_body,
        grid=(N // bn,),
        in_specs=[
            pl.BlockSpec((NC, bn, D), lambda i: (0, i, 0)),
            pl.BlockSpec((NW, bn, 1), lambda i: (0, i, 0)),
        ],
        out_specs=pl.BlockSpec((bn, D), lambda i: (i, 0)),
        out_shape=jax.ShapeDtypeStruct((N, D), jnp.float32),
    )(outp, dr_p)


def kernel(x, adj, W, b):
    s = adj[0].astype(jnp.int32)
    r = adj[1].astype(jnp.int32)

    degp = _sc_degrees(s, r).reshape(NW, 2, N_PAD)
    ds_p = degp[:, 0, :, None]
    dr_p = degp[:, 1, :, None]

    h = _tc_linear(x, W, b.reshape(1, D), ds_p)

    pad = E_PAD - E
    s_p = jnp.concatenate([s, jnp.zeros((pad,), jnp.int32)]).reshape(
        NW, G, 128)
    r_p = jnp.concatenate([r, jnp.full((pad,), TRASH, jnp.int32)]).reshape(
        NW, G, 128)

    outp = _sc_segsum(h, s_p, r_p)
    y = _tc_final(outp, dr_p)
    return (y, adj)
